# baseline (speedup 1.0000x reference)
import functools

import jax
import jax.numpy as jnp
from jax import lax
from jax.experimental import pallas as pl
from jax.experimental.pallas import tpu as pltpu
from jax.experimental.pallas import tpu_sc as plsc

_NB = 2


@functools.lru_cache(maxsize=None)
def _build(B, L, V, D, NC, NS):
    NW = NC * NS
    n_per_w = B * L // NW
    b_per_w = B // NW
    chunk = _NB * L
    n_chunks = n_per_w // chunk
    assert chunk <= 128 and n_chunks % 4 == 0 and n_chunks >= 8
    mesh = plsc.VectorSubcoreMesh(core_axis_name="c", subcore_axis_name="s")

    @functools.partial(
        pl.kernel,
        mesh=mesh,
        compiler_params=pltpu.CompilerParams(use_tc_tiling_on_sc=True),
        out_type=jax.ShapeDtypeStruct((B, L, D), jnp.float32),
        scratch_types=[
            pltpu.VMEM((n_chunks, _NB * L), jnp.int32),
            pltpu.VMEM((4, _NB * L, D), jnp.float32),
            pltpu.SemaphoreType.DMA,
            pltpu.SemaphoreType.DMA,
            pltpu.SemaphoreType.DMA,
            pltpu.SemaphoreType.DMA,
            pltpu.SemaphoreType.DMA,
            pltpu.SemaphoreType.DMA,
            pltpu.SemaphoreType.DMA,
            pltpu.SemaphoreType.DMA,
        ],
    )
    def k(idx_hbm, table_hbm, out_hbm, idx_v, rows_v,
          g0, g1, g2, g3, s0, s1, s2, s3):
        wid = lax.axis_index("s") * NC + lax.axis_index("c")
        batch0 = wid * b_per_w
        gsem = (g0, g1, g2, g3)
        ssem = (s0, s1, s2, s3)
        pltpu.sync_copy(idx_hbm.at[wid], idx_v)

        def start_gather(j, b):
            pltpu.async_copy(
                table_hbm.at[idx_v.at[j]],
                rows_v.at[b],
                gsem[b],
            )

        def wait_gather(j, b):
            pltpu.make_async_copy(
                table_hbm.at[idx_v.at[j]], rows_v.at[b], gsem[b]
            ).wait()

        def start_store(j, b):
            for u in range(_NB):
                pltpu.async_copy(
                    rows_v.at[b, pl.ds(u * L, L)],
                    out_hbm.at[batch0 + j * _NB + u],
                    ssem[b],
                )

        def wait_store(b):
            for u in range(_NB):
                pltpu.make_async_copy(
                    rows_v.at[b, pl.ds(0, L)], out_hbm.at[0], ssem[b]
                ).wait()

        start_gather(0, 0)
        start_gather(1, 1)
        start_gather(2, 2)
        wait_gather(0, 0)
        start_store(0, 0)
        start_gather(3, 3)

        def body(i, carry):
            for u in range(4):
                j = 1 + 4 * i + u
                b = (1 + u) % 4
                wait_gather(j, b)
                start_store(j, b)
                wait_store((b + 3) % 4)
                start_gather(j + 3, (b + 3) % 4)
            return carry

        lax.fori_loop(0, (n_chunks - 4) // 4, body, 0)

        for j in (n_chunks - 3, n_chunks - 2, n_chunks - 1):
            b = j % 4
            wait_gather(j, b)
            start_store(j, b)
            wait_store((b + 3) % 4)
        wait_store((n_chunks - 1) % 4)

    return k


def kernel(indices, table):
    B, L = indices.shape
    V, D = table.shape
    info = plsc.get_sparse_core_info()
    NW = info.num_cores * info.num_subcores
    n_chunks = B // (NW * _NB)
    idx_flat = indices.reshape(NW, n_chunks, _NB * L).astype(jnp.int32)
    k = _build(B, L, V, D, info.num_cores, info.num_subcores)
    return k(idx_flat, table)


# speedup vs baseline: 1.7564x; 1.7564x over previous
import functools

import jax
import jax.numpy as jnp
from jax import lax
from jax.experimental import pallas as pl
from jax.experimental.pallas import tpu as pltpu
from jax.experimental.pallas import tpu_sc as plsc

_CHUNK = 128
_NBUF = 6


@functools.lru_cache(maxsize=None)
def _build(N, V, D, NC, NS):
    NW = NC * NS
    n_per_w = N // NW
    n_chunks = n_per_w // _CHUNK
    n_steady = ((n_chunks - _NBUF) // _NBUF) * _NBUF
    assert n_chunks >= 2 * _NBUF
    mesh = plsc.VectorSubcoreMesh(core_axis_name="c", subcore_axis_name="s")

    @functools.partial(
        pl.kernel,
        mesh=mesh,
        out_type=jax.ShapeDtypeStruct((N, D), jnp.float32),
        scratch_types=[
            pltpu.VMEM((n_chunks, _CHUNK), jnp.int32),
            pltpu.VMEM((_NBUF, _CHUNK, D), jnp.float32),
        ]
        + [pltpu.SemaphoreType.DMA] * (2 * _NBUF),
    )
    def k(idx_hbm, table_hbm, out_hbm, idx_v, rows_v, *sems):
        wid = lax.axis_index("s") * NC + lax.axis_index("c")
        chunk0 = wid * n_chunks
        gsem = sems[:_NBUF]
        ssem = sems[_NBUF:]
        pltpu.sync_copy(idx_hbm.at[wid], idx_v)

        def start_gather(j, b):
            pltpu.async_copy(table_hbm.at[idx_v.at[j]], rows_v.at[b], gsem[b])

        def wait_gather(j, b):
            pltpu.make_async_copy(
                table_hbm.at[idx_v.at[j]], rows_v.at[b], gsem[b]
            ).wait()

        def start_store(j, b):
            pltpu.async_copy(
                rows_v.at[b],
                out_hbm.at[pl.ds((chunk0 + j) * _CHUNK, _CHUNK)],
                ssem[b],
            )

        def wait_store(b):
            pltpu.make_async_copy(
                rows_v.at[b], out_hbm.at[pl.ds(0, _CHUNK)], ssem[b]
            ).wait()

        for b in range(_NBUF - 1):
            start_gather(b, b)
        wait_gather(0, 0)
        start_store(0, 0)
        start_gather(_NBUF - 1, _NBUF - 1)

        def step(j, b):
            wait_gather(j, b)
            start_store(j, b)
            wait_store((b + _NBUF - 1) % _NBUF)
            if isinstance(j, int) and j + _NBUF - 1 >= n_chunks:
                return
            start_gather(j + _NBUF - 1, (b + _NBUF - 1) % _NBUF)

        def body(i, carry):
            for u in range(_NBUF):
                j = 1 + _NBUF * i + u
                step(j, (1 + u) % _NBUF)
            return carry

        lax.fori_loop(0, n_steady // _NBUF, body, 0)

        for j in range(n_steady + 1, n_chunks):
            step(j, j % _NBUF)
        wait_store((n_chunks - 1) % _NBUF)

    return k


def kernel(indices, table):
    B, L = indices.shape
    V, D = table.shape
    N = B * L
    info = plsc.get_sparse_core_info()
    NW = info.num_cores * info.num_subcores
    idx_t = indices.T.reshape(NW, N // (NW * _CHUNK), _CHUNK).astype(jnp.int32)
    k = _build(N, V, D, info.num_cores, info.num_subcores)
    out = k(idx_t, table)
    return out.reshape(L, B, D).transpose(1, 0, 2)
